# Initial kernel scaffold; baseline (speedup 1.0000x reference)
import jax
import jax.numpy as jnp
from jax.experimental import pallas as pl


def kernel(x, w1, w2, w3, g1, b1, g2, b2, g3, b3):
    raise NotImplementedError("write your pallas kernel here")


# speedup vs baseline: 15.4171x; 15.4171x over previous
import functools

import jax
import jax.numpy as jnp
from jax import lax
from jax.experimental import pallas as pl
from jax.experimental.pallas import tpu as pltpu

EPS = 1e-5


def _stats(y, ps_ref, pq_ref):
    ps_ref[...] = jnp.broadcast_to(jnp.sum(y, axis=0, keepdims=True),
                                   ps_ref.shape)
    pq_ref[...] = jnp.broadcast_to(jnp.sum(y * y, axis=0, keepdims=True),
                                   pq_ref.shape)


def _conv1_kernel(x_ref, w_ref, y_ref, ps_ref, pq_ref):
    y = jnp.dot(x_ref[...], w_ref[...], preferred_element_type=jnp.float32)
    y_ref[...] = y.astype(y_ref.dtype)
    _stats(y, ps_ref, pq_ref)


def _conv2_kernel(y1_ref, sc_ref, sh_ref, w2_ref, y2_ref, ps_ref, pq_ref,
                  pad_ref, tap_ref, *, imgs, hh, wh, c):
    a = jnp.maximum(y1_ref[...].astype(jnp.float32) * sc_ref[...]
                    + sh_ref[...], 0.0).astype(pad_ref.dtype)
    pad_ref[...] = jnp.zeros(pad_ref.shape, pad_ref.dtype)
    for ph in range(2):
        for pw in range(2):
            pad_ref[:, 1 - ph, 1 - pw, ph:ph + hh, pw:pw + wh, :] = \
                a[:, ph, pw]
    tap_ref[:, :, wh:, :] = jnp.zeros((imgs, hh, 2, 9 * c), tap_ref.dtype)
    for kh in range(3):
        for kw in range(3):
            t = pad_ref[:, kh % 2, kw % 2,
                        kh // 2:kh // 2 + hh, kw // 2:kw // 2 + wh, :]
            tap_ref[:, :, :wh, (3 * kh + kw) * c:(3 * kh + kw + 1) * c] = t
    tt = tap_ref[...].reshape(imgs * hh * (wh + 2), 9 * c)
    y2 = jnp.dot(tt, w2_ref[...], preferred_element_type=jnp.float32)
    y2_ref[...] = y2.astype(y2_ref.dtype)
    _stats(y2, ps_ref, pq_ref)


def _conv3_kernel(y2_ref, sc_ref, sh_ref, w_ref, y_ref, ps_ref, pq_ref, *,
                  wp, wh):
    a = jnp.maximum(y2_ref[...].astype(jnp.float32) * sc_ref[...]
                    + sh_ref[...], 0.0)
    col = lax.broadcasted_iota(jnp.int32, a.shape, 0) % wp
    a = jnp.where(col < wh, a, 0.0)
    y = jnp.dot(a.astype(w_ref.dtype), w_ref[...],
                preferred_element_type=jnp.float32)
    y_ref[...] = y.astype(y_ref.dtype)
    _stats(y, ps_ref, pq_ref)


def _bn_kernel(y_ref, sc_ref, sh_ref, o_ref):
    o_ref[...] = (y_ref[...].astype(jnp.float32) * sc_ref[...]
                  + sh_ref[...]).astype(o_ref.dtype)


def _collapse(p, g):
    c = p.shape[-1]
    return p.reshape(g, 8, c)[:, 0, :].sum(axis=0)


def _scale_shift(psum, pq, g, count, gamma, beta):
    mean = _collapse(psum, g) / count
    var = jnp.maximum(_collapse(pq, g) / count - mean * mean, 0.0)
    scale = gamma * lax.rsqrt(var + EPS)
    shift = beta - mean * scale
    return scale.reshape(1, -1), shift.reshape(1, -1)


_PAR = pltpu.CompilerParams(dimension_semantics=("parallel",))


def kernel(x, w1, w2, w3, g1, b1, g2, b2, g3, b3):
    n, cin, h, w = x.shape
    cw = w1.shape[1]
    cout = w3.shape[1]
    hh, wh = h // 2, w // 2
    wp = wh + 2
    m1 = n * h * w
    m2p = n * hh * wp
    bf = jnp.bfloat16

    w1b = w1.astype(bf)
    w2b = w2.astype(bf)
    w3b = w3.astype(bf)
    xb = jnp.transpose(x.reshape(n, cin, hh, 2, wh, 2),
                       (0, 3, 5, 2, 4, 1)).astype(bf)

    g_a = 8
    tm_a = m1 // g_a
    y1, s1p, q1p = pl.pallas_call(
        _conv1_kernel,
        out_shape=(jax.ShapeDtypeStruct((m1, cw), bf),
                   jax.ShapeDtypeStruct((8 * g_a, cw), jnp.float32),
                   jax.ShapeDtypeStruct((8 * g_a, cw), jnp.float32)),
        grid=(g_a,),
        in_specs=[pl.BlockSpec((tm_a, cin), lambda i: (i, 0)),
                  pl.BlockSpec((cin, cw), lambda i: (0, 0))],
        out_specs=(pl.BlockSpec((tm_a, cw), lambda i: (i, 0)),
                   pl.BlockSpec((8, cw), lambda i: (i, 0)),
                   pl.BlockSpec((8, cw), lambda i: (i, 0))),
        compiler_params=_PAR,
    )(xb.reshape(m1, cin), w1b)
    sc1, sh1 = _scale_shift(s1p, q1p, g_a, m1, g1, b1)

    imgs = 4
    g_b = n // imgs
    kfn = functools.partial(_conv2_kernel, imgs=imgs, hh=hh, wh=wh, c=cw)
    y2, s2p, q2p = pl.pallas_call(
        kfn,
        out_shape=(jax.ShapeDtypeStruct((m2p, cw), bf),
                   jax.ShapeDtypeStruct((8 * g_b, cw), jnp.float32),
                   jax.ShapeDtypeStruct((8 * g_b, cw), jnp.float32)),
        grid=(g_b,),
        in_specs=[pl.BlockSpec((imgs, 2, 2, hh, wh, cw),
                               lambda i: (i, 0, 0, 0, 0, 0)),
                  pl.BlockSpec((1, cw), lambda i: (0, 0)),
                  pl.BlockSpec((1, cw), lambda i: (0, 0)),
                  pl.BlockSpec((9 * cw, cw), lambda i: (0, 0))],
        out_specs=(pl.BlockSpec((imgs * hh * wp, cw), lambda i: (i, 0)),
                   pl.BlockSpec((8, cw), lambda i: (i, 0)),
                   pl.BlockSpec((8, cw), lambda i: (i, 0))),
        scratch_shapes=[pltpu.VMEM((imgs, 2, 2, hh + 1, wh + 1, cw), bf),
                        pltpu.VMEM((imgs, hh, wp, 9 * cw), bf)],
        compiler_params=_PAR,
    )(y1.reshape(n, 2, 2, hh, wh, cw), sc1, sh1, w2b)
    m2 = n * hh * wh
    sc2, sh2 = _scale_shift(s2p, q2p, g_b, m2, g2, b2)

    g_c = 8
    tm_c = m2p // g_c
    y3, s3p, q3p = pl.pallas_call(
        functools.partial(_conv3_kernel, wp=wp, wh=wh),
        out_shape=(jax.ShapeDtypeStruct((m2p, cout), jnp.float32),
                   jax.ShapeDtypeStruct((8 * g_c, cout), jnp.float32),
                   jax.ShapeDtypeStruct((8 * g_c, cout), jnp.float32)),
        grid=(g_c,),
        in_specs=[pl.BlockSpec((tm_c, cw), lambda i: (i, 0)),
                  pl.BlockSpec((1, cw), lambda i: (0, 0)),
                  pl.BlockSpec((1, cw), lambda i: (0, 0)),
                  pl.BlockSpec((cw, cout), lambda i: (0, 0))],
        out_specs=(pl.BlockSpec((tm_c, cout), lambda i: (i, 0)),
                   pl.BlockSpec((8, cout), lambda i: (i, 0)),
                   pl.BlockSpec((8, cout), lambda i: (i, 0))),
        compiler_params=_PAR,
    )(y2, sc2, sh2, w3b)
    sc3, sh3 = _scale_shift(s3p, q3p, g_c, m2, g3, b3)

    out = pl.pallas_call(
        _bn_kernel,
        out_shape=jax.ShapeDtypeStruct((m2p, cout), jnp.float32),
        grid=(g_c,),
        in_specs=[pl.BlockSpec((tm_c, cout), lambda i: (i, 0)),
                  pl.BlockSpec((1, cout), lambda i: (0, 0)),
                  pl.BlockSpec((1, cout), lambda i: (0, 0))],
        out_specs=pl.BlockSpec((tm_c, cout), lambda i: (i, 0)),
        compiler_params=_PAR,
    )(y3, sc3, sh3)

    out = out.reshape(n, hh, wp, cout)[:, :, :wh, :]
    return jnp.transpose(out, (0, 3, 1, 2))
